# Initial kernel scaffold; baseline (speedup 1.0000x reference)
import jax
import jax.numpy as jnp
from jax.experimental import pallas as pl


def kernel(x, edge_index, concentration, W1, b1, W2, b2, fc_W, fc_b):
    raise NotImplementedError("write your pallas kernel here")


# speedup vs baseline: 13.6885x; 13.6885x over previous
import functools

import jax
import jax.numpy as jnp
from jax import lax
from jax.experimental import pallas as pl
from jax.experimental.pallas import tpu as pltpu
from jax.experimental.pallas import tpu_sc as plsc

NC = 2
NS = 16
NW = NC * NS
LANES = 16
CHUNK = 128
DEGW = 8


def _sc_mesh():
    return plsc.VectorSubcoreMesh(core_axis_name="c", subcore_axis_name="s",
                                  num_cores=NC, num_subcores=NS)


def _make_deg(nch, npad):
    rpt = npad // NS
    nch2 = nch // 2

    @functools.partial(
        pl.kernel,
        out_type=jax.ShapeDtypeStruct((NC, npad, DEGW), jnp.float32),
        mesh=_sc_mesh(),
        scratch_types=[
            pltpu.VMEM((nch2, CHUNK), jnp.int32),
            pltpu.VMEM((CHUNK, DEGW), jnp.float32),
            pltpu.VMEM_SHARED((npad, DEGW), jnp.float32),
        ],
        compiler_params=pltpu.CompilerParams(use_tc_tiling_on_sc=False),
    )
    def degk(dstp, ones, zeros8, out, idx_v, ones_v, acc):
        c = lax.axis_index("c")
        s = lax.axis_index("s")
        pltpu.sync_copy(dstp.at[s, pl.ds(c * nch2, nch2)], idx_v)
        pltpu.sync_copy(ones, ones_v)
        pltpu.sync_copy(zeros8, acc.at[pl.ds(s * rpt, rpt)])
        plsc.subcore_barrier()

        def body(j, carry):
            pltpu.sync_copy(ones_v, acc.at[idx_v.at[j]], add=True)
            return carry

        lax.fori_loop(0, nch2, body, 0)
        plsc.subcore_barrier()
        pltpu.sync_copy(acc.at[pl.ds(s * rpt, rpt)],
                        out.at[c, pl.ds(s * rpt, rpt)])

    return degk


def _make_scatter_rows(n, dh, nch, npad):
    rpt = npad // NS

    @functools.partial(
        pl.kernel,
        out_type=jax.ShapeDtypeStruct((NC, npad, dh), jnp.float32),
        mesh=_sc_mesh(),
        scratch_types=[
            pltpu.VMEM((nch, CHUNK), jnp.int32),
            pltpu.VMEM((nch, CHUNK), jnp.int32),
            pltpu.VMEM((CHUNK, dh), jnp.float32),
            pltpu.VMEM((CHUNK, dh), jnp.float32),
            pltpu.VMEM_SHARED((npad, dh), jnp.float32),
            pltpu.SemaphoreType.DMA,
            pltpu.SemaphoreType.DMA,
        ],
        compiler_params=pltpu.CompilerParams(use_tc_tiling_on_sc=False),
    )
    def scatk(tab_lo, tab_hi, srcp, dstp, zeros, out,
              src_v, dst_v, rows0, rows1, acc, sem0, sem1):
        c = lax.axis_index("c")
        s = lax.axis_index("s")
        pltpu.sync_copy(srcp.at[s], src_v)
        pltpu.sync_copy(dstp.at[s], dst_v)
        pltpu.sync_copy(zeros, acc.at[pl.ds(s * rpt, rpt)])
        plsc.subcore_barrier()

        def run(table):
            pltpu.async_copy(table.at[src_v.at[0]], rows0, sem0)
            pltpu.async_copy(table.at[src_v.at[1]], rows1, sem1)

            def pair(j2, carry):
                for b in range(2):
                    rows = rows0 if b == 0 else rows1
                    sem = sem0 if b == 0 else sem1
                    j = j2 * 2 + b
                    pltpu.make_async_copy(table.at[src_v.at[j]], rows,
                                          sem).wait()
                    pltpu.sync_copy(rows, acc.at[dst_v.at[j]], add=True)

                    @pl.when(j + 2 < nch)
                    def _issue():
                        pltpu.async_copy(table.at[src_v.at[j + 2]], rows, sem)

                return carry

            lax.fori_loop(0, nch // 2, pair, 0)

        @pl.when(c == 0)
        def _lo():
            run(tab_lo)

        @pl.when(c == 1)
        def _hi():
            run(tab_hi)

        plsc.subcore_barrier()
        pltpu.sync_copy(acc.at[pl.ds(s * rpt, rpt)],
                        out.at[c, pl.ds(s * rpt, rpt)])

    return scatk


def _make_readout(n, nch):
    nch2 = nch // 2

    @functools.partial(
        pl.kernel,
        out_type=jax.ShapeDtypeStruct((NC, NS, LANES), jnp.float32),
        mesh=_sc_mesh(),
        scratch_types=[
            pltpu.VMEM((nch2, CHUNK), jnp.int32),
            pltpu.VMEM((nch2, CHUNK), jnp.int32),
            pltpu.VMEM((n,), jnp.float32),
            pltpu.VMEM((LANES,), jnp.float32),
        ],
        compiler_params=pltpu.CompilerParams(needs_layout_passes=False),
    )
    def readk(p_hbm, srcp, dstp, out, src_v, dst_v, p_v, acc_v):
        c = lax.axis_index("c")
        s = lax.axis_index("s")
        pltpu.sync_copy(srcp.at[s, pl.ds(c * nch2, nch2)], src_v)
        pltpu.sync_copy(dstp.at[s, pl.ds(c * nch2, nch2)], dst_v)
        pltpu.sync_copy(p_hbm, p_v)

        def body(j, acc):
            for i in range(CHUNK // LANES):
                idx = src_v[j, pl.ds(i * LANES, LANES)]
                dvec = dst_v[j, pl.ds(i * LANES, LANES)]
                pv = plsc.load_gather(p_v, [idx])
                acc = acc + jnp.where(dvec == 0, pv,
                                      jnp.zeros((LANES,), jnp.float32))
            return acc

        acc = lax.fori_loop(0, nch2, body,
                            jnp.zeros((LANES,), jnp.float32))
        acc_v[...] = acc
        pltpu.sync_copy(acc_v, out.at[c, s])

    return readk


_BR = 1000


def _mm_body(x_ref, w_ref, o_ref):
    o_ref[...] = jnp.dot(x_ref[...], w_ref[...],
                         preferred_element_type=jnp.float32)


def _matmul(x, w):
    n, d = x.shape
    nb = n // _BR
    return pl.pallas_call(
        _mm_body,
        grid=(nb,),
        in_specs=[pl.BlockSpec((_BR, d), lambda i: (i, 0)),
                  pl.BlockSpec((d, w.shape[1]), lambda i: (0, 0))],
        out_specs=pl.BlockSpec((_BR, w.shape[1]), lambda i: (i, 0)),
        out_shape=jax.ShapeDtypeStruct((n, w.shape[1]), jnp.float32),
    )(x, w)


def _prep_body(d0_ref, d1_ref, h_ref, dinv_ref, lo_ref, hi_ref):
    dv = lax.rsqrt(d0_ref[...] + d1_ref[...] + 1.0)
    dinv_ref[...] = dv
    hs = h_ref[...] * dv
    dh = hs.shape[1] // 2
    lo_ref[...] = hs[:, :dh]
    hi_ref[...] = hs[:, dh:]


def _prep(d0, d1, h1):
    n, d = h1.shape
    nb = n // _BR
    dh = d // 2
    return pl.pallas_call(
        _prep_body,
        grid=(nb,),
        in_specs=[pl.BlockSpec((_BR, 1), lambda i: (i, 0)),
                  pl.BlockSpec((_BR, 1), lambda i: (i, 0)),
                  pl.BlockSpec((_BR, d), lambda i: (i, 0))],
        out_specs=[pl.BlockSpec((_BR, 1), lambda i: (i, 0)),
                   pl.BlockSpec((_BR, dh), lambda i: (i, 0)),
                   pl.BlockSpec((_BR, dh), lambda i: (i, 0))],
        out_shape=[jax.ShapeDtypeStruct((n, 1), jnp.float32),
                   jax.ShapeDtypeStruct((n, dh), jnp.float32),
                   jax.ShapeDtypeStruct((n, dh), jnp.float32)],
    )(d0, d1, h1)


def _layer_body(alo_ref, ahi_ref, h_ref, dinv_ref, b_ref, w_ref,
                h2_ref, lo_ref, hi_ref):
    dv = dinv_ref[...]
    a = jnp.concatenate([alo_ref[...], ahi_ref[...]], axis=1)
    z = jnp.maximum(a * dv + h_ref[...] * (dv * dv) + b_ref[...], 0.0)
    h2 = jnp.dot(z, w_ref[...], preferred_element_type=jnp.float32)
    h2_ref[...] = h2
    hs = h2 * dv
    dh = hs.shape[1] // 2
    lo_ref[...] = hs[:, :dh]
    hi_ref[...] = hs[:, dh:]


def _layer(alo, ahi, h1, dinv, b, w):
    n, d = h1.shape
    nb = n // _BR
    dh = d // 2
    return pl.pallas_call(
        _layer_body,
        grid=(nb,),
        in_specs=[pl.BlockSpec((_BR, dh), lambda i: (i, 0)),
                  pl.BlockSpec((_BR, dh), lambda i: (i, 0)),
                  pl.BlockSpec((_BR, d), lambda i: (i, 0)),
                  pl.BlockSpec((_BR, 1), lambda i: (i, 0)),
                  pl.BlockSpec((1, d), lambda i: (0, 0)),
                  pl.BlockSpec((d, d), lambda i: (0, 0))],
        out_specs=[pl.BlockSpec((_BR, d), lambda i: (i, 0)),
                   pl.BlockSpec((_BR, dh), lambda i: (i, 0)),
                   pl.BlockSpec((_BR, dh), lambda i: (i, 0))],
        out_shape=[jax.ShapeDtypeStruct((n, d), jnp.float32),
                   jax.ShapeDtypeStruct((n, dh), jnp.float32),
                   jax.ShapeDtypeStruct((n, dh), jnp.float32)],
    )(alo, ahi, h1, dinv, b, w)


def _read_body(alo_ref, ahi_ref, h_ref, dinv_ref, b_ref, fw_ref,
               p_ref, q_ref):
    dv = dinv_ref[...]
    a = jnp.concatenate([alo_ref[...], ahi_ref[...]], axis=1)
    z = jnp.maximum(a * dv + h_ref[...] * (dv * dv) + b_ref[...], 0.0)
    fw = fw_ref[...]
    d = h_ref.shape[1]
    q_ref[...] = jnp.dot(z, fw[0:d, :], preferred_element_type=jnp.float32)
    p_ref[...] = jnp.dot(z, fw[d:2 * d, :], preferred_element_type=jnp.float32)


def _read(alo, ahi, h2, dinv, b, fw):
    n, d = h2.shape
    nb = n // _BR
    dh = d // 2
    return pl.pallas_call(
        _read_body,
        grid=(nb,),
        in_specs=[pl.BlockSpec((_BR, dh), lambda i: (i, 0)),
                  pl.BlockSpec((_BR, dh), lambda i: (i, 0)),
                  pl.BlockSpec((_BR, d), lambda i: (i, 0)),
                  pl.BlockSpec((_BR, 1), lambda i: (i, 0)),
                  pl.BlockSpec((1, d), lambda i: (0, 0)),
                  pl.BlockSpec((2 * d, 1), lambda i: (0, 0))],
        out_specs=[pl.BlockSpec((_BR, 1), lambda i: (i, 0)),
                   pl.BlockSpec((_BR, 1), lambda i: (i, 0))],
        out_shape=[jax.ShapeDtypeStruct((n, 1), jnp.float32),
                   jax.ShapeDtypeStruct((n, 1), jnp.float32)],
    )(alo, ahi, h2, dinv, b, fw)


def kernel(x, edge_index, concentration, W1, b1, W2, b2, fc_W, fc_b):
    n, d = x.shape
    dh = d // 2
    e = edge_index.shape[1]
    nch = 16 * (-(-e // (NS * CHUNK * 16)))
    ept = nch * CHUNK
    epad = ept * NS
    npad = -(-(n + 1) // (NS * 8)) * (NS * 8)
    rpt = npad // NS

    src = edge_index[0]
    dst = edge_index[1]
    padn = epad - e
    srcp = jnp.concatenate(
        [src, jnp.zeros((padn,), jnp.int32)]).reshape(NS, nch, CHUNK)
    dstp = jnp.concatenate(
        [dst, jnp.full((padn,), n, jnp.int32)]).reshape(NS, nch, CHUNK)
    ones8 = jnp.ones((CHUNK, DEGW), jnp.float32)
    zeros8 = jnp.zeros((rpt, DEGW), jnp.float32)
    zerosd = jnp.zeros((rpt, dh), jnp.float32)

    degk = _make_deg(nch, npad)
    scatk = _make_scatter_rows(n, dh, nch, npad)
    readk = _make_readout(n, nch)

    degout = degk(dstp, ones8, zeros8)
    h1 = _matmul(x, W1)

    d0 = degout[0, :n, 0:1]
    d1 = degout[1, :n, 0:1]
    dinv, hs1lo, hs1hi = _prep(d0, d1, h1)

    acc1 = scatk(hs1lo, hs1hi, srcp, dstp, zerosd)
    h2, hs2lo, hs2hi = _layer(acc1[0, :n], acc1[1, :n], h1, dinv,
                              b1.reshape(1, d), W2)
    acc2 = scatk(hs2lo, hs2hi, srcp, dstp, zerosd)
    p, q = _read(acc2[0, :n], acc2[1, :n], h2, dinv,
                 b2.reshape(1, d), fc_W)

    pr = readk(p.reshape(-1), srcp, dstp)
    segp = jnp.sum(pr)
    cnt = jnp.maximum(degout[0, 0, 0] + degout[1, 0, 0], 1.0)
    return q[0] + segp / cnt + fc_b


# speedup vs baseline: 13.7063x; 1.0013x over previous
import functools

import jax
import jax.numpy as jnp
from jax import lax
from jax.experimental import pallas as pl
from jax.experimental.pallas import tpu as pltpu
from jax.experimental.pallas import tpu_sc as plsc

NC = 2
NS = 16
NW = NC * NS
LANES = 16
CHUNK = 128
DEGW = 8


def _sc_mesh():
    return plsc.VectorSubcoreMesh(core_axis_name="c", subcore_axis_name="s",
                                  num_cores=NC, num_subcores=NS)


def _make_deg(nch, npad):
    rpt = npad // NS
    nch2 = nch // 2

    @functools.partial(
        pl.kernel,
        out_type=jax.ShapeDtypeStruct((NC, npad, DEGW), jnp.float32),
        mesh=_sc_mesh(),
        scratch_types=[
            pltpu.VMEM((nch2, CHUNK), jnp.int32),
            pltpu.VMEM((CHUNK, DEGW), jnp.float32),
            pltpu.VMEM_SHARED((npad, DEGW), jnp.float32),
        ],
        compiler_params=pltpu.CompilerParams(use_tc_tiling_on_sc=False),
    )
    def degk(dstp, ones, zeros8, out, idx_v, ones_v, acc):
        c = lax.axis_index("c")
        s = lax.axis_index("s")
        pltpu.sync_copy(dstp.at[s, pl.ds(c * nch2, nch2)], idx_v)
        pltpu.sync_copy(ones, ones_v)
        pltpu.sync_copy(zeros8, acc.at[pl.ds(s * rpt, rpt)])
        plsc.subcore_barrier()

        def body(j, carry):
            pltpu.sync_copy(ones_v, acc.at[idx_v.at[j]], add=True)
            return carry

        lax.fori_loop(0, nch2, body, 0)
        plsc.subcore_barrier()
        pltpu.sync_copy(acc.at[pl.ds(s * rpt, rpt)],
                        out.at[c, pl.ds(s * rpt, rpt)])

    return degk


def _make_scatter_rows(n, dh, nch, npad):
    rpt = npad // NS

    @functools.partial(
        pl.kernel,
        out_type=jax.ShapeDtypeStruct((NC, npad, dh), jnp.float32),
        mesh=_sc_mesh(),
        scratch_types=[
            pltpu.VMEM((nch, CHUNK), jnp.int32),
            pltpu.VMEM((nch, CHUNK), jnp.int32),
            pltpu.VMEM((CHUNK, dh), jnp.float32),
            pltpu.VMEM((CHUNK, dh), jnp.float32),
            pltpu.VMEM((CHUNK, dh), jnp.float32),
            pltpu.VMEM((CHUNK, dh), jnp.float32),
            pltpu.VMEM_SHARED((npad, dh), jnp.float32),
            [pltpu.SemaphoreType.DMA] * 4,
            [pltpu.SemaphoreType.DMA] * 4,
        ],
        compiler_params=pltpu.CompilerParams(use_tc_tiling_on_sc=False),
    )
    def scatk(tab_lo, tab_hi, srcp, dstp, zeros, out,
              src_v, dst_v, rows0, rows1, rows2, rows3, acc, gsems, ssems):
        c = lax.axis_index("c")
        s = lax.axis_index("s")
        pltpu.sync_copy(srcp.at[s], src_v)
        pltpu.sync_copy(dstp.at[s], dst_v)
        pltpu.sync_copy(zeros, acc.at[pl.ds(s * rpt, rpt)])
        plsc.subcore_barrier()

        bufs = (rows0, rows1, rows2, rows3)

        def run(table):
            pltpu.async_copy(table.at[src_v.at[0]], bufs[0], gsems[0])
            pltpu.async_copy(table.at[src_v.at[1]], bufs[1], gsems[1])

            def quad(j4, carry):
                for u in range(4):
                    j = j4 * 4 + u
                    rows = bufs[u]
                    pltpu.make_async_copy(table.at[src_v.at[j]], rows,
                                          gsems[u]).wait()
                    pltpu.async_copy(rows, acc.at[dst_v.at[j]], ssems[u],
                                     add=True)
                    u2 = (u + 2) % 4
                    rows2_ = bufs[u2]

                    @pl.when(j >= 2)
                    def _drain():
                        pltpu.make_async_copy(
                            rows2_, acc.at[dst_v.at[j - 2]],
                            ssems[u2]).wait()

                    @pl.when(j + 2 < nch)
                    def _issue():
                        pltpu.async_copy(table.at[src_v.at[j + 2]], rows2_,
                                         gsems[u2])

                return carry

            lax.fori_loop(0, nch // 4, quad, 0)
            pltpu.make_async_copy(bufs[(nch - 2) % 4],
                                  acc.at[dst_v.at[nch - 2]],
                                  ssems[(nch - 2) % 4]).wait()
            pltpu.make_async_copy(bufs[(nch - 1) % 4],
                                  acc.at[dst_v.at[nch - 1]],
                                  ssems[(nch - 1) % 4]).wait()

        @pl.when(c == 0)
        def _lo():
            run(tab_lo)

        @pl.when(c == 1)
        def _hi():
            run(tab_hi)

        plsc.subcore_barrier()
        pltpu.sync_copy(acc.at[pl.ds(s * rpt, rpt)],
                        out.at[c, pl.ds(s * rpt, rpt)])

    return scatk


def _make_readout(n, nch):
    nch2 = nch // 2

    @functools.partial(
        pl.kernel,
        out_type=jax.ShapeDtypeStruct((NC, NS, LANES), jnp.float32),
        mesh=_sc_mesh(),
        scratch_types=[
            pltpu.VMEM((nch2, CHUNK), jnp.int32),
            pltpu.VMEM((nch2, CHUNK), jnp.int32),
            pltpu.VMEM((n,), jnp.float32),
            pltpu.VMEM((LANES,), jnp.float32),
        ],
        compiler_params=pltpu.CompilerParams(needs_layout_passes=False),
    )
    def readk(p_hbm, srcp, dstp, out, src_v, dst_v, p_v, acc_v):
        c = lax.axis_index("c")
        s = lax.axis_index("s")
        pltpu.sync_copy(srcp.at[s, pl.ds(c * nch2, nch2)], src_v)
        pltpu.sync_copy(dstp.at[s, pl.ds(c * nch2, nch2)], dst_v)
        pltpu.sync_copy(p_hbm, p_v)

        def body(j, acc):
            for i in range(CHUNK // LANES):
                idx = src_v[j, pl.ds(i * LANES, LANES)]
                dvec = dst_v[j, pl.ds(i * LANES, LANES)]
                pv = plsc.load_gather(p_v, [idx])
                acc = acc + jnp.where(dvec == 0, pv,
                                      jnp.zeros((LANES,), jnp.float32))
            return acc

        acc = lax.fori_loop(0, nch2, body,
                            jnp.zeros((LANES,), jnp.float32))
        acc_v[...] = acc
        pltpu.sync_copy(acc_v, out.at[c, s])

    return readk


_BR = 1000


def _mm_body(x_ref, w_ref, o_ref):
    o_ref[...] = jnp.dot(x_ref[...], w_ref[...],
                         preferred_element_type=jnp.float32)


def _matmul(x, w):
    n, d = x.shape
    nb = n // _BR
    return pl.pallas_call(
        _mm_body,
        grid=(nb,),
        in_specs=[pl.BlockSpec((_BR, d), lambda i: (i, 0)),
                  pl.BlockSpec((d, w.shape[1]), lambda i: (0, 0))],
        out_specs=pl.BlockSpec((_BR, w.shape[1]), lambda i: (i, 0)),
        out_shape=jax.ShapeDtypeStruct((n, w.shape[1]), jnp.float32),
    )(x, w)


def _prep_body(d0_ref, d1_ref, h_ref, dinv_ref, lo_ref, hi_ref):
    dv = lax.rsqrt(d0_ref[...] + d1_ref[...] + 1.0)
    dinv_ref[...] = dv
    hs = h_ref[...] * dv
    dh = hs.shape[1] // 2
    lo_ref[...] = hs[:, :dh]
    hi_ref[...] = hs[:, dh:]


def _prep(d0, d1, h1):
    n, d = h1.shape
    nb = n // _BR
    dh = d // 2
    return pl.pallas_call(
        _prep_body,
        grid=(nb,),
        in_specs=[pl.BlockSpec((_BR, 1), lambda i: (i, 0)),
                  pl.BlockSpec((_BR, 1), lambda i: (i, 0)),
                  pl.BlockSpec((_BR, d), lambda i: (i, 0))],
        out_specs=[pl.BlockSpec((_BR, 1), lambda i: (i, 0)),
                   pl.BlockSpec((_BR, dh), lambda i: (i, 0)),
                   pl.BlockSpec((_BR, dh), lambda i: (i, 0))],
        out_shape=[jax.ShapeDtypeStruct((n, 1), jnp.float32),
                   jax.ShapeDtypeStruct((n, dh), jnp.float32),
                   jax.ShapeDtypeStruct((n, dh), jnp.float32)],
    )(d0, d1, h1)


def _layer_body(alo_ref, ahi_ref, h_ref, dinv_ref, b_ref, w_ref,
                h2_ref, lo_ref, hi_ref):
    dv = dinv_ref[...]
    a = jnp.concatenate([alo_ref[...], ahi_ref[...]], axis=1)
    z = jnp.maximum(a * dv + h_ref[...] * (dv * dv) + b_ref[...], 0.0)
    h2 = jnp.dot(z, w_ref[...], preferred_element_type=jnp.float32)
    h2_ref[...] = h2
    hs = h2 * dv
    dh = hs.shape[1] // 2
    lo_ref[...] = hs[:, :dh]
    hi_ref[...] = hs[:, dh:]


def _layer(alo, ahi, h1, dinv, b, w):
    n, d = h1.shape
    nb = n // _BR
    dh = d // 2
    return pl.pallas_call(
        _layer_body,
        grid=(nb,),
        in_specs=[pl.BlockSpec((_BR, dh), lambda i: (i, 0)),
                  pl.BlockSpec((_BR, dh), lambda i: (i, 0)),
                  pl.BlockSpec((_BR, d), lambda i: (i, 0)),
                  pl.BlockSpec((_BR, 1), lambda i: (i, 0)),
                  pl.BlockSpec((1, d), lambda i: (0, 0)),
                  pl.BlockSpec((d, d), lambda i: (0, 0))],
        out_specs=[pl.BlockSpec((_BR, d), lambda i: (i, 0)),
                   pl.BlockSpec((_BR, dh), lambda i: (i, 0)),
                   pl.BlockSpec((_BR, dh), lambda i: (i, 0))],
        out_shape=[jax.ShapeDtypeStruct((n, d), jnp.float32),
                   jax.ShapeDtypeStruct((n, dh), jnp.float32),
                   jax.ShapeDtypeStruct((n, dh), jnp.float32)],
    )(alo, ahi, h1, dinv, b, w)


def _read_body(alo_ref, ahi_ref, h_ref, dinv_ref, b_ref, fw_ref,
               p_ref, q_ref):
    dv = dinv_ref[...]
    a = jnp.concatenate([alo_ref[...], ahi_ref[...]], axis=1)
    z = jnp.maximum(a * dv + h_ref[...] * (dv * dv) + b_ref[...], 0.0)
    fw = fw_ref[...]
    d = h_ref.shape[1]
    q_ref[...] = jnp.dot(z, fw[0:d, :], preferred_element_type=jnp.float32)
    p_ref[...] = jnp.dot(z, fw[d:2 * d, :], preferred_element_type=jnp.float32)


def _read(alo, ahi, h2, dinv, b, fw):
    n, d = h2.shape
    nb = n // _BR
    dh = d // 2
    return pl.pallas_call(
        _read_body,
        grid=(nb,),
        in_specs=[pl.BlockSpec((_BR, dh), lambda i: (i, 0)),
                  pl.BlockSpec((_BR, dh), lambda i: (i, 0)),
                  pl.BlockSpec((_BR, d), lambda i: (i, 0)),
                  pl.BlockSpec((_BR, 1), lambda i: (i, 0)),
                  pl.BlockSpec((1, d), lambda i: (0, 0)),
                  pl.BlockSpec((2 * d, 1), lambda i: (0, 0))],
        out_specs=[pl.BlockSpec((_BR, 1), lambda i: (i, 0)),
                   pl.BlockSpec((_BR, 1), lambda i: (i, 0))],
        out_shape=[jax.ShapeDtypeStruct((n, 1), jnp.float32),
                   jax.ShapeDtypeStruct((n, 1), jnp.float32)],
    )(alo, ahi, h2, dinv, b, fw)


def kernel(x, edge_index, concentration, W1, b1, W2, b2, fc_W, fc_b):
    n, d = x.shape
    dh = d // 2
    e = edge_index.shape[1]
    nch = 16 * (-(-e // (NS * CHUNK * 16)))
    ept = nch * CHUNK
    epad = ept * NS
    npad = -(-(n + 1) // (NS * 8)) * (NS * 8)
    rpt = npad // NS

    src = edge_index[0]
    dst = edge_index[1]
    padn = epad - e
    srcp = jnp.concatenate(
        [src, jnp.zeros((padn,), jnp.int32)]).reshape(NS, nch, CHUNK)
    dstp = jnp.concatenate(
        [dst, jnp.full((padn,), n, jnp.int32)]).reshape(NS, nch, CHUNK)
    ones8 = jnp.ones((CHUNK, DEGW), jnp.float32)
    zeros8 = jnp.zeros((rpt, DEGW), jnp.float32)
    zerosd = jnp.zeros((rpt, dh), jnp.float32)

    degk = _make_deg(nch, npad)
    scatk = _make_scatter_rows(n, dh, nch, npad)
    readk = _make_readout(n, nch)

    degout = degk(dstp, ones8, zeros8)
    h1 = _matmul(x, W1)

    d0 = degout[0, :n, 0:1]
    d1 = degout[1, :n, 0:1]
    dinv, hs1lo, hs1hi = _prep(d0, d1, h1)

    acc1 = scatk(hs1lo, hs1hi, srcp, dstp, zerosd)
    h2, hs2lo, hs2hi = _layer(acc1[0, :n], acc1[1, :n], h1, dinv,
                              b1.reshape(1, d), W2)
    acc2 = scatk(hs2lo, hs2hi, srcp, dstp, zerosd)
    p, q = _read(acc2[0, :n], acc2[1, :n], h2, dinv,
                 b2.reshape(1, d), fc_W)

    pr = readk(p.reshape(-1), srcp, dstp)
    segp = jnp.sum(pr)
    cnt = jnp.maximum(degout[0, 0, 0] + degout[1, 0, 0], 1.0)
    return q[0] + segp / cnt + fc_b


# speedup vs baseline: 13.9673x; 1.0190x over previous
import functools

import jax
import jax.numpy as jnp
from jax import lax
from jax.experimental import pallas as pl
from jax.experimental.pallas import tpu as pltpu
from jax.experimental.pallas import tpu_sc as plsc

NC = 2
NS = 16
NW = NC * NS
LANES = 16
CHUNK = 128
DEGW = 8


def _sc_mesh():
    return plsc.VectorSubcoreMesh(core_axis_name="c", subcore_axis_name="s",
                                  num_cores=NC, num_subcores=NS)


def _make_deg(nch, npad):
    rpt = npad // NS
    nch2 = nch // 2

    @functools.partial(
        pl.kernel,
        out_type=jax.ShapeDtypeStruct((NC, npad, DEGW), jnp.float32),
        mesh=_sc_mesh(),
        scratch_types=[
            pltpu.VMEM((nch2, CHUNK), jnp.int32),
            pltpu.VMEM((CHUNK, DEGW), jnp.float32),
            pltpu.VMEM_SHARED((npad, DEGW), jnp.float32),
        ],
        compiler_params=pltpu.CompilerParams(use_tc_tiling_on_sc=False),
    )
    def degk(dstp, ones, zeros8, out, idx_v, ones_v, acc):
        c = lax.axis_index("c")
        s = lax.axis_index("s")
        pltpu.sync_copy(dstp.at[s, pl.ds(c * nch2, nch2)], idx_v)
        pltpu.sync_copy(ones, ones_v)
        pltpu.sync_copy(zeros8, acc.at[pl.ds(s * rpt, rpt)])
        plsc.subcore_barrier()

        def body(j, carry):
            pltpu.sync_copy(ones_v, acc.at[idx_v.at[j]], add=True)
            return carry

        lax.fori_loop(0, nch2, body, 0)
        plsc.subcore_barrier()
        pltpu.sync_copy(acc.at[pl.ds(s * rpt, rpt)],
                        out.at[c, pl.ds(s * rpt, rpt)])

    return degk


QCH = 2


def _make_scatter_rows(n, dh, nch, npad):
    rpt = npad // NS
    nq = nch // QCH

    @functools.partial(
        pl.kernel,
        out_type=jax.ShapeDtypeStruct((NC, npad, dh), jnp.float32),
        mesh=_sc_mesh(),
        scratch_types=[
            pltpu.VMEM((nq, QCH * CHUNK), jnp.int32),
            pltpu.VMEM((nq, QCH * CHUNK), jnp.int32),
            pltpu.VMEM((QCH * CHUNK, dh), jnp.float32),
            pltpu.VMEM((QCH * CHUNK, dh), jnp.float32),
            pltpu.VMEM_SHARED((npad, dh), jnp.float32),
            pltpu.SemaphoreType.DMA,
            pltpu.SemaphoreType.DMA,
        ],
        compiler_params=pltpu.CompilerParams(use_tc_tiling_on_sc=False),
    )
    def scatk(tab_lo, tab_hi, srcp, dstp, zeros, out,
              src_v, dst_v, rows0, rows1, acc, sem0, sem1):
        c = lax.axis_index("c")
        s = lax.axis_index("s")
        pltpu.sync_copy(srcp.at[s], src_v)
        pltpu.sync_copy(dstp.at[s], dst_v)
        pltpu.sync_copy(zeros, acc.at[pl.ds(s * rpt, rpt)])
        plsc.subcore_barrier()

        def run(table):
            pltpu.async_copy(table.at[src_v.at[0]], rows0, sem0)
            pltpu.async_copy(table.at[src_v.at[1]], rows1, sem1)

            def pair(q2, carry):
                for b in range(2):
                    rows = rows0 if b == 0 else rows1
                    sem = sem0 if b == 0 else sem1
                    q = q2 * 2 + b
                    pltpu.make_async_copy(table.at[src_v.at[q]], rows,
                                          sem).wait()
                    pltpu.sync_copy(rows, acc.at[dst_v.at[q]], add=True)

                    @pl.when(q + 2 < nq)
                    def _issue():
                        pltpu.async_copy(table.at[src_v.at[q + 2]], rows, sem)

                return carry

            lax.fori_loop(0, nq // 2, pair, 0)

        @pl.when(c == 0)
        def _lo():
            run(tab_lo)

        @pl.when(c == 1)
        def _hi():
            run(tab_hi)

        plsc.subcore_barrier()
        pltpu.sync_copy(acc.at[pl.ds(s * rpt, rpt)],
                        out.at[c, pl.ds(s * rpt, rpt)])

    return scatk


def _make_readout(n, nch):
    nch2 = nch // 2

    @functools.partial(
        pl.kernel,
        out_type=jax.ShapeDtypeStruct((NC, NS, LANES), jnp.float32),
        mesh=_sc_mesh(),
        scratch_types=[
            pltpu.VMEM((nch2, CHUNK), jnp.int32),
            pltpu.VMEM((nch2, CHUNK), jnp.int32),
            pltpu.VMEM((n,), jnp.float32),
            pltpu.VMEM((LANES,), jnp.float32),
        ],
        compiler_params=pltpu.CompilerParams(needs_layout_passes=False),
    )
    def readk(p_hbm, srcp, dstp, out, src_v, dst_v, p_v, acc_v):
        c = lax.axis_index("c")
        s = lax.axis_index("s")
        pltpu.sync_copy(srcp.at[s, pl.ds(c * nch2, nch2)], src_v)
        pltpu.sync_copy(dstp.at[s, pl.ds(c * nch2, nch2)], dst_v)
        pltpu.sync_copy(p_hbm, p_v)

        def body(j, acc):
            for i in range(CHUNK // LANES):
                idx = src_v[j, pl.ds(i * LANES, LANES)]
                dvec = dst_v[j, pl.ds(i * LANES, LANES)]
                pv = plsc.load_gather(p_v, [idx])
                acc = acc + jnp.where(dvec == 0, pv,
                                      jnp.zeros((LANES,), jnp.float32))
            return acc

        acc = lax.fori_loop(0, nch2, body,
                            jnp.zeros((LANES,), jnp.float32))
        acc_v[...] = acc
        pltpu.sync_copy(acc_v, out.at[c, s])

    return readk


_BR = 1000


def _mm_body(x_ref, w_ref, o_ref):
    o_ref[...] = jnp.dot(x_ref[...], w_ref[...],
                         preferred_element_type=jnp.float32)


def _matmul(x, w):
    n, d = x.shape
    nb = n // _BR
    return pl.pallas_call(
        _mm_body,
        grid=(nb,),
        in_specs=[pl.BlockSpec((_BR, d), lambda i: (i, 0)),
                  pl.BlockSpec((d, w.shape[1]), lambda i: (0, 0))],
        out_specs=pl.BlockSpec((_BR, w.shape[1]), lambda i: (i, 0)),
        out_shape=jax.ShapeDtypeStruct((n, w.shape[1]), jnp.float32),
    )(x, w)


def _prep_body(d0_ref, d1_ref, h_ref, dinv_ref, lo_ref, hi_ref):
    dv = lax.rsqrt(d0_ref[...] + d1_ref[...] + 1.0)
    dinv_ref[...] = dv
    hs = h_ref[...] * dv
    dh = hs.shape[1] // 2
    lo_ref[...] = hs[:, :dh]
    hi_ref[...] = hs[:, dh:]


def _prep(d0, d1, h1):
    n, d = h1.shape
    nb = n // _BR
    dh = d // 2
    return pl.pallas_call(
        _prep_body,
        grid=(nb,),
        in_specs=[pl.BlockSpec((_BR, 1), lambda i: (i, 0)),
                  pl.BlockSpec((_BR, 1), lambda i: (i, 0)),
                  pl.BlockSpec((_BR, d), lambda i: (i, 0))],
        out_specs=[pl.BlockSpec((_BR, 1), lambda i: (i, 0)),
                   pl.BlockSpec((_BR, dh), lambda i: (i, 0)),
                   pl.BlockSpec((_BR, dh), lambda i: (i, 0))],
        out_shape=[jax.ShapeDtypeStruct((n, 1), jnp.float32),
                   jax.ShapeDtypeStruct((n, dh), jnp.float32),
                   jax.ShapeDtypeStruct((n, dh), jnp.float32)],
    )(d0, d1, h1)


def _layer_body(alo_ref, ahi_ref, h_ref, dinv_ref, b_ref, w_ref,
                h2_ref, lo_ref, hi_ref):
    dv = dinv_ref[...]
    a = jnp.concatenate([alo_ref[...], ahi_ref[...]], axis=1)
    z = jnp.maximum(a * dv + h_ref[...] * (dv * dv) + b_ref[...], 0.0)
    h2 = jnp.dot(z, w_ref[...], preferred_element_type=jnp.float32)
    h2_ref[...] = h2
    hs = h2 * dv
    dh = hs.shape[1] // 2
    lo_ref[...] = hs[:, :dh]
    hi_ref[...] = hs[:, dh:]


def _layer(alo, ahi, h1, dinv, b, w):
    n, d = h1.shape
    nb = n // _BR
    dh = d // 2
    return pl.pallas_call(
        _layer_body,
        grid=(nb,),
        in_specs=[pl.BlockSpec((_BR, dh), lambda i: (i, 0)),
                  pl.BlockSpec((_BR, dh), lambda i: (i, 0)),
                  pl.BlockSpec((_BR, d), lambda i: (i, 0)),
                  pl.BlockSpec((_BR, 1), lambda i: (i, 0)),
                  pl.BlockSpec((1, d), lambda i: (0, 0)),
                  pl.BlockSpec((d, d), lambda i: (0, 0))],
        out_specs=[pl.BlockSpec((_BR, d), lambda i: (i, 0)),
                   pl.BlockSpec((_BR, dh), lambda i: (i, 0)),
                   pl.BlockSpec((_BR, dh), lambda i: (i, 0))],
        out_shape=[jax.ShapeDtypeStruct((n, d), jnp.float32),
                   jax.ShapeDtypeStruct((n, dh), jnp.float32),
                   jax.ShapeDtypeStruct((n, dh), jnp.float32)],
    )(alo, ahi, h1, dinv, b, w)


def _read_body(alo_ref, ahi_ref, h_ref, dinv_ref, b_ref, fw_ref,
               p_ref, q_ref):
    dv = dinv_ref[...]
    a = jnp.concatenate([alo_ref[...], ahi_ref[...]], axis=1)
    z = jnp.maximum(a * dv + h_ref[...] * (dv * dv) + b_ref[...], 0.0)
    fw = fw_ref[...]
    d = h_ref.shape[1]
    q_ref[...] = jnp.dot(z, fw[0:d, :], preferred_element_type=jnp.float32)
    p_ref[...] = jnp.dot(z, fw[d:2 * d, :], preferred_element_type=jnp.float32)


def _read(alo, ahi, h2, dinv, b, fw):
    n, d = h2.shape
    nb = n // _BR
    dh = d // 2
    return pl.pallas_call(
        _read_body,
        grid=(nb,),
        in_specs=[pl.BlockSpec((_BR, dh), lambda i: (i, 0)),
                  pl.BlockSpec((_BR, dh), lambda i: (i, 0)),
                  pl.BlockSpec((_BR, d), lambda i: (i, 0)),
                  pl.BlockSpec((_BR, 1), lambda i: (i, 0)),
                  pl.BlockSpec((1, d), lambda i: (0, 0)),
                  pl.BlockSpec((2 * d, 1), lambda i: (0, 0))],
        out_specs=[pl.BlockSpec((_BR, 1), lambda i: (i, 0)),
                   pl.BlockSpec((_BR, 1), lambda i: (i, 0))],
        out_shape=[jax.ShapeDtypeStruct((n, 1), jnp.float32),
                   jax.ShapeDtypeStruct((n, 1), jnp.float32)],
    )(alo, ahi, h2, dinv, b, fw)


def kernel(x, edge_index, concentration, W1, b1, W2, b2, fc_W, fc_b):
    n, d = x.shape
    dh = d // 2
    e = edge_index.shape[1]
    nch = 16 * (-(-e // (NS * CHUNK * 16)))
    ept = nch * CHUNK
    epad = ept * NS
    npad = -(-(n + 1) // (NS * 8)) * (NS * 8)
    rpt = npad // NS

    src = edge_index[0]
    dst = edge_index[1]
    padn = epad - e
    srcp = jnp.concatenate(
        [src, jnp.zeros((padn,), jnp.int32)]).reshape(NS, nch, CHUNK)
    dstp = jnp.concatenate(
        [dst, jnp.full((padn,), n, jnp.int32)]).reshape(NS, nch, CHUNK)
    ones8 = jnp.ones((CHUNK, DEGW), jnp.float32)
    zeros8 = jnp.zeros((rpt, DEGW), jnp.float32)
    zerosd = jnp.zeros((rpt, dh), jnp.float32)

    degk = _make_deg(nch, npad)
    scatk = _make_scatter_rows(n, dh, nch, npad)
    readk = _make_readout(n, nch)

    degout = degk(dstp, ones8, zeros8)
    h1 = _matmul(x, W1)

    d0 = degout[0, :n, 0:1]
    d1 = degout[1, :n, 0:1]
    dinv, hs1lo, hs1hi = _prep(d0, d1, h1)

    src4 = srcp.reshape(NS, nch // QCH, QCH * CHUNK)
    dst4 = dstp.reshape(NS, nch // QCH, QCH * CHUNK)
    acc1 = scatk(hs1lo, hs1hi, src4, dst4, zerosd)
    h2, hs2lo, hs2hi = _layer(acc1[0, :n], acc1[1, :n], h1, dinv,
                              b1.reshape(1, d), W2)
    acc2 = scatk(hs2lo, hs2hi, src4, dst4, zerosd)
    p, q = _read(acc2[0, :n], acc2[1, :n], h2, dinv,
                 b2.reshape(1, d), fc_W)

    pr = readk(p.reshape(-1), srcp, dstp)
    segp = jnp.sum(pr)
    cnt = jnp.maximum(degout[0, 0, 0] + degout[1, 0, 0], 1.0)
    return q[0] + segp / cnt + fc_b
